# Initial kernel scaffold; baseline (speedup 1.0000x reference)
import jax
import jax.numpy as jnp
from jax.experimental import pallas as pl


def kernel(input, input_timestamp, edge_index, edge_type, edge_weight, step_len, emb_weight, bias1, bias2):
    raise NotImplementedError("write your pallas kernel here")


# speedup vs baseline: 7.8927x; 7.8927x over previous
import functools

import jax
import jax.numpy as jnp
from jax import lax
from jax.experimental import pallas as pl
from jax.experimental.pallas import tpu as pltpu
from jax.experimental.pallas import tpu_sc as plsc

N_NODES = 10000
N_EDGES = 320000
D = 128
DH = 64
NC = 2
NT = 16
N_PAD = 10240
E_PAD = 327680
BK = 128
EB = E_PAD // NT // BK
CH = 8
NCH = EB // CH
T = 12800
TBK = 80
TB = T // NT // TBK
ROWS_PT = N_PAD // NT


def _rsqrt16(v):
    i = lax.bitcast_convert_type(v, jnp.int32)
    i = 0x5F3759DF - lax.shift_right_logical(i, 1)
    y = lax.bitcast_convert_type(i, jnp.float32)
    vh = v * 0.5
    for _ in range(3):
        y = y * (1.5 - vh * y * y)
    return jnp.where(v > 0.0, y, 0.0)


def _sc_body(xs, row3, col3, ew4, et4, tok3, b12,
             out, h1s, h2s,
             row_v, col_v, coef_v, ewc, etc, dis_v, tok_v, gbuf,
             b1_v, b2_v, zdeg, deg_s, h_s):
    c = lax.axis_index("c")
    s = lax.axis_index("s")
    z16 = jnp.zeros((16,), jnp.float32)

    def zero_gbuf():
        def zr(j, _):
            for k in range(DH // 16):
                gbuf[j, pl.ds(k * 16, 16)] = z16
            return 0

        lax.fori_loop(0, BK, zr, 0)

    def zero_h():
        def zh(m, _):
            pltpu.sync_copy(gbuf, h_s.at[pl.ds(s * ROWS_PT + m * BK, BK)])
            return 0

        lax.fori_loop(0, ROWS_PT // BK, zh, 0)

    def dump_h(dst, bias_v):
        def dm(m, _):
            base = s * ROWS_PT + m * BK
            pltpu.sync_copy(h_s.at[pl.ds(base, BK)], gbuf)

            def brow(j, _):
                for k in range(DH // 16):
                    sl = pl.ds(k * 16, 16)
                    gbuf[j, sl] = gbuf[j, sl] + bias_v[sl]
                return 0

            lax.fori_loop(0, BK, brow, 0)
            pltpu.sync_copy(gbuf, dst.at[pl.ds(c * N_PAD + base, BK)])
            return 0

        lax.fori_loop(0, ROWS_PT // BK, dm, 0)

    def conv_layer(table):
        def blk(b, _):
            pltpu.sync_copy(table.at[row_v.at[b]], gbuf)

            def srow(g, _):
                cvec = coef_v[b, pl.ds(g * 16, 16)]
                for jj in range(16):
                    cf = cvec[jj]
                    j = g * 16 + jj
                    for k in range(DH // 16):
                        sl = pl.ds(k * 16, 16)
                        gbuf[j, sl] = gbuf[j, sl] * cf
                return 0

            lax.fori_loop(0, BK // 16, srow, 0)
            pltpu.sync_copy(gbuf, h_s.at[col_v.at[b]], add=True)
            return 0

        lax.fori_loop(0, EB, blk, 0)

    pltpu.sync_copy(row3.at[s], row_v)
    pltpu.sync_copy(col3.at[s], col_v)
    pltpu.sync_copy(tok3.at[s], tok_v)
    pltpu.sync_copy(b12.at[pl.ds(c * DH, DH)], b1_v)
    pltpu.sync_copy(b12.at[pl.ds((2 + c) * DH, DH)], b2_v)

    def zd(i, _):
        zdeg[pl.ds(i * 16, 16)] = z16
        return 0

    lax.fori_loop(0, ROWS_PT // 16, zd, 0)
    zero_gbuf()
    pltpu.sync_copy(zdeg, deg_s.at[pl.ds(s * ROWS_PT, ROWS_PT)])
    zero_h()
    plsc.subcore_barrier()

    def deg_ch(ch, _):
        pltpu.sync_copy(ew4.at[s * NCH + ch], ewc)
        for b8 in range(CH):
            pltpu.sync_copy(ewc.at[b8], deg_s.at[row_v.at[ch * CH + b8]],
                            add=True)
        return 0

    lax.fori_loop(0, NCH, deg_ch, 0)
    plsc.subcore_barrier()

    pltpu.sync_copy(deg_s, dis_v)

    def dis_blk(i, _):
        sl = pl.ds(i * 16, 16)
        dis_v[sl] = _rsqrt16(dis_v[sl])
        return 0

    lax.fori_loop(0, N_PAD // 16, dis_blk, 0)

    def coef_ch(ch, _):
        pltpu.sync_copy(ew4.at[s * NCH + ch], ewc)
        pltpu.sync_copy(et4.at[s * NCH + ch], etc)
        for b8 in range(CH):
            b = ch * CH + b8
            for k in range(BK // 16):
                sl = pl.ds(k * 16, 16)
                d1 = plsc.load_gather(dis_v, [row_v[b, sl]])
                d2 = plsc.load_gather(dis_v, [col_v[b, sl]])
                coef_v[b, sl] = (d1 * d2 * ewc[b8, sl]
                                 * (0.9 - 0.8 * etc[b8, sl]))
        return 0

    lax.fori_loop(0, NCH, coef_ch, 0)

    def shift_blk(b, _):
        for k in range(BK // 16):
            sl = pl.ds(k * 16, 16)
            row_v[b, sl] = row_v[b, sl] + c * N_PAD
        return 0

    lax.fori_loop(0, EB, shift_blk, 0)

    def tok_shift(b, _):
        for k in range(TBK // 16):
            sl = pl.ds(k * 16, 16)
            tok_v[b, sl] = tok_v[b, sl] + c * N_PAD
        return 0

    lax.fori_loop(0, TB, tok_shift, 0)

    conv_layer(xs)
    plsc.subcore_barrier()

    dump_h(h1s, b1_v)
    zero_gbuf()
    zero_h()
    plsc.subcore_barrier()

    conv_layer(h1s)
    plsc.subcore_barrier()

    dump_h(h2s, b2_v)
    plsc.subcore_barrier()

    def tok_blk(tb, _):
        pltpu.sync_copy(h2s.at[tok_v.at[tb]], gbuf.at[pl.ds(0, TBK)])
        pltpu.sync_copy(
            gbuf.at[pl.ds(0, TBK)],
            out.at[pl.ds(c * T + s * (TB * TBK) + tb * TBK, TBK)])
        return 0

    lax.fori_loop(0, TB, tok_blk, 0)


@jax.jit
def _run(xs, row3, col3, ew4, et4, tok3, b12):
    f = pl.kernel(
        _sc_body,
        out_type=[
            jax.ShapeDtypeStruct((NC * T, DH), jnp.float32),
            jax.ShapeDtypeStruct((NC * N_PAD, DH), jnp.float32),
            jax.ShapeDtypeStruct((NC * N_PAD, DH), jnp.float32),
        ],
        mesh=plsc.VectorSubcoreMesh(core_axis_name="c", subcore_axis_name="s"),
        compiler_params=pltpu.CompilerParams(
            needs_layout_passes=False, use_tc_tiling_on_sc=False),
        scratch_types=[
            pltpu.VMEM((EB, BK), jnp.int32),
            pltpu.VMEM((EB, BK), jnp.int32),
            pltpu.VMEM((EB, BK), jnp.float32),
            pltpu.VMEM((CH, BK), jnp.float32),
            pltpu.VMEM((CH, BK), jnp.float32),
            pltpu.VMEM((N_PAD,), jnp.float32),
            pltpu.VMEM((TB, TBK), jnp.int32),
            pltpu.VMEM((BK, DH), jnp.float32),
            pltpu.VMEM((DH,), jnp.float32),
            pltpu.VMEM((DH,), jnp.float32),
            pltpu.VMEM((ROWS_PT,), jnp.float32),
            pltpu.VMEM_SHARED((N_PAD,), jnp.float32),
            pltpu.VMEM_SHARED((N_PAD, DH), jnp.float32),
        ],
    )
    return f(xs, row3, col3, ew4, et4, tok3, b12)


def kernel(input, input_timestamp, edge_index, edge_type, edge_weight,
           step_len, emb_weight, bias1, bias2):
    del input_timestamp, step_len
    B, L = input.shape
    row = edge_index[0].astype(jnp.int32)
    col = edge_index[1].astype(jnp.int32)
    ew = edge_weight.astype(jnp.float32)
    et = edge_type.astype(jnp.float32)
    pad = E_PAD - row.shape[0]
    row3 = jnp.pad(row, (0, pad)).reshape(NT, EB, BK)
    col3 = jnp.pad(col, (0, pad)).reshape(NT, EB, BK)
    ew4 = jnp.pad(ew, (0, pad)).reshape(NT * NCH, CH, BK)
    et4 = jnp.pad(et, (0, pad)).reshape(NT * NCH, CH, BK)
    xs = jnp.zeros((NC * N_PAD, DH), jnp.float32)
    xs = xs.at[:N_NODES].set(emb_weight[:, :DH])
    xs = xs.at[N_PAD:N_PAD + N_NODES].set(emb_weight[:, DH:])
    tok3 = input.reshape(-1).astype(jnp.int32).reshape(NT, TB, TBK)
    b12 = jnp.concatenate([bias1[:DH], bias1[DH:], bias2[:DH], bias2[DH:]])
    out, _h1, _h2 = _run(xs, row3, col3, ew4, et4, tok3, b12)
    return jnp.concatenate([out[:T], out[T:]], axis=-1).reshape(B, L, D)


# speedup vs baseline: 10.7534x; 1.3624x over previous
import functools

import jax
import jax.numpy as jnp
from jax import lax
from jax.experimental import pallas as pl
from jax.experimental.pallas import tpu as pltpu
from jax.experimental.pallas import tpu_sc as plsc

N_NODES = 10000
N_EDGES = 320000
D = 128
DH = 64
NC = 2
NT = 16
N_PAD = 10240
E_PAD = 327680
BK = 128
EB = E_PAD // NT // BK
CH = 4
NCH = EB // CH
T = 12800
TBK = 80
TB = T // NT // TBK
ROWS_PT = N_PAD // NT


def _rsqrt16(v):
    i = lax.bitcast_convert_type(v, jnp.int32)
    i = 0x5F3759DF - lax.shift_right_logical(i, 1)
    y = lax.bitcast_convert_type(i, jnp.float32)
    vh = v * 0.5
    for _ in range(3):
        y = y * (1.5 - vh * y * y)
    return jnp.where(v > 0.0, y, 0.0)


def _sc_body(xs, row3, col3, ew4, et4, tok4, b12,
             out, h1s, h2s,
             row_v, col_v, coef_v, ewc, etc, dis_v, tokc, gbuf, gbb,
             b_v, zdeg, sga, sgb, deg_s, h_s):
    c = lax.axis_index("c")
    s = lax.axis_index("s")
    z16 = jnp.zeros((16,), jnp.float32)

    def zero_gbuf():
        def zr(j, _):
            for k in range(DH // 16):
                gbuf[j, pl.ds(k * 16, 16)] = z16
            return 0

        lax.fori_loop(0, BK, zr, 0)

    def zero_h():
        def zh(m, _):
            pltpu.sync_copy(gbuf, h_s.at[pl.ds(s * ROWS_PT + m * BK, BK)])
            return 0

        lax.fori_loop(0, ROWS_PT // BK, zh, 0)

    def dump_h(dst, bias_v):
        def dm(m, _):
            base = s * ROWS_PT + m * BK
            pltpu.sync_copy(h_s.at[pl.ds(base, BK)], gbuf)

            def brow(j, _):
                for k in range(DH // 16):
                    sl = pl.ds(k * 16, 16)
                    gbuf[j, sl] = gbuf[j, sl] + bias_v[sl]
                return 0

            lax.fori_loop(0, BK, brow, 0)
            pltpu.sync_copy(gbuf, dst.at[pl.ds(c * N_PAD + base, BK)])
            return 0

        lax.fori_loop(0, ROWS_PT // BK, dm, 0)

    def conv_layer(table):
        def scale(buf, b):
            def srow(g, _):
                cvec = coef_v[b, pl.ds(g * 16, 16)]
                for jj in range(16):
                    cf = cvec[jj]
                    j = g * 16 + jj
                    for k in range(DH // 16):
                        sl = pl.ds(k * 16, 16)
                        buf[j, sl] = buf[j, sl] * cf
                return 0

            lax.fori_loop(0, BK // 16, srow, 0)

        pltpu.async_copy(table.at[row_v.at[0]], gbuf, sga)

        def pair(g, _):
            b0 = 2 * g
            pltpu.async_copy(table.at[row_v.at[b0 + 1]], gbb, sgb)
            pltpu.make_async_copy(table.at[row_v.at[b0]], gbuf, sga).wait()
            scale(gbuf, b0)
            pltpu.sync_copy(gbuf, h_s.at[col_v.at[b0]], add=True)

            @pl.when(g < EB // 2 - 1)
            def _():
                pltpu.async_copy(table.at[row_v.at[b0 + 2]], gbuf, sga)

            pltpu.make_async_copy(table.at[row_v.at[b0 + 1]], gbb, sgb).wait()
            scale(gbb, b0 + 1)
            pltpu.sync_copy(gbb, h_s.at[col_v.at[b0 + 1]], add=True)
            return 0

        lax.fori_loop(0, EB // 2, pair, 0)

    pltpu.sync_copy(row3.at[s], row_v)
    pltpu.sync_copy(col3.at[s], col_v)
    pltpu.sync_copy(b12.at[pl.ds(c * DH, DH)], b_v)
    zdeg[pl.ds(0, 16)] = z16
    zero_gbuf()

    def zd(i, _):
        pltpu.sync_copy(zdeg, deg_s.at[pl.ds(s * ROWS_PT + i * 16, 16)])
        return 0

    lax.fori_loop(0, ROWS_PT // 16, zd, 0)
    zero_h()
    plsc.subcore_barrier()

    def deg_ch(ch, _):
        pltpu.sync_copy(ew4.at[s * NCH + ch], ewc)
        for b8 in range(CH):
            pltpu.sync_copy(ewc.at[b8], deg_s.at[row_v.at[ch * CH + b8]],
                            add=True)
        return 0

    lax.fori_loop(0, NCH, deg_ch, 0)
    plsc.subcore_barrier()

    pltpu.sync_copy(deg_s, dis_v)

    def dis_blk(i, _):
        sl = pl.ds(i * 16, 16)
        dis_v[sl] = _rsqrt16(dis_v[sl])
        return 0

    lax.fori_loop(0, N_PAD // 16, dis_blk, 0)

    def coef_ch(ch, _):
        pltpu.sync_copy(ew4.at[s * NCH + ch], ewc)
        pltpu.sync_copy(et4.at[s * NCH + ch], etc)
        for b8 in range(CH):
            b = ch * CH + b8
            for k in range(BK // 16):
                sl = pl.ds(k * 16, 16)
                d1 = plsc.load_gather(dis_v, [row_v[b, sl]])
                d2 = plsc.load_gather(dis_v, [col_v[b, sl]])
                coef_v[b, sl] = (d1 * d2 * ewc[b8, sl]
                                 * (0.9 - 0.8 * etc[b8, sl]))
        return 0

    lax.fori_loop(0, NCH, coef_ch, 0)

    def shift_blk(b, _):
        for k in range(BK // 16):
            sl = pl.ds(k * 16, 16)
            row_v[b, sl] = row_v[b, sl] + c * N_PAD
        return 0

    lax.fori_loop(0, EB, shift_blk, 0)

    conv_layer(xs)
    plsc.subcore_barrier()

    dump_h(h1s, b_v)
    zero_gbuf()
    zero_h()
    plsc.subcore_barrier()

    conv_layer(h1s)
    plsc.subcore_barrier()

    pltpu.sync_copy(b12.at[pl.ds((2 + c) * DH, DH)], b_v)
    dump_h(h2s, b_v)
    plsc.subcore_barrier()

    def tok_blk(tb, _):
        pltpu.sync_copy(tok4.at[s * TB + tb], tokc)
        for k in range(TBK // 16):
            sl = pl.ds(k * 16, 16)
            tokc[sl] = tokc[sl] + c * N_PAD
        pltpu.sync_copy(h2s.at[tokc], gbuf.at[pl.ds(0, TBK)])
        pltpu.sync_copy(
            gbuf.at[pl.ds(0, TBK)],
            out.at[pl.ds(c * T + s * (TB * TBK) + tb * TBK, TBK)])
        return 0

    lax.fori_loop(0, TB, tok_blk, 0)


@jax.jit
def _run(xs, row3, col3, ew4, et4, tok4, b12):
    f = pl.kernel(
        _sc_body,
        out_type=[
            jax.ShapeDtypeStruct((NC * T, DH), jnp.float32),
            jax.ShapeDtypeStruct((NC * N_PAD, DH), jnp.float32),
            jax.ShapeDtypeStruct((NC * N_PAD, DH), jnp.float32),
        ],
        mesh=plsc.VectorSubcoreMesh(core_axis_name="c", subcore_axis_name="s"),
        compiler_params=pltpu.CompilerParams(
            needs_layout_passes=False, use_tc_tiling_on_sc=False),
        scratch_types=[
            pltpu.VMEM((EB, BK), jnp.int32),
            pltpu.VMEM((EB, BK), jnp.int32),
            pltpu.VMEM((EB, BK), jnp.float32),
            pltpu.VMEM((CH, BK), jnp.float32),
            pltpu.VMEM((CH, BK), jnp.float32),
            pltpu.VMEM((N_PAD,), jnp.float32),
            pltpu.VMEM((TBK,), jnp.int32),
            pltpu.VMEM((BK, DH), jnp.float32),
            pltpu.VMEM((BK, DH), jnp.float32),
            pltpu.VMEM((DH,), jnp.float32),
            pltpu.VMEM((16,), jnp.float32),
            pltpu.SemaphoreType.DMA,
            pltpu.SemaphoreType.DMA,
            pltpu.VMEM_SHARED((N_PAD,), jnp.float32),
            pltpu.VMEM_SHARED((N_PAD, DH), jnp.float32),
        ],
    )
    return f(xs, row3, col3, ew4, et4, tok4, b12)


def kernel(input, input_timestamp, edge_index, edge_type, edge_weight,
           step_len, emb_weight, bias1, bias2):
    del input_timestamp, step_len
    B, L = input.shape
    row = edge_index[0].astype(jnp.int32)
    col = edge_index[1].astype(jnp.int32)
    ew = edge_weight.astype(jnp.float32)
    et = edge_type.astype(jnp.float32)
    pad = E_PAD - row.shape[0]
    row3 = jnp.pad(row, (0, pad)).reshape(NT, EB, BK)
    col3 = jnp.pad(col, (0, pad)).reshape(NT, EB, BK)
    ew4 = jnp.pad(ew, (0, pad)).reshape(NT * NCH, CH, BK)
    et4 = jnp.pad(et, (0, pad)).reshape(NT * NCH, CH, BK)
    xs = jnp.zeros((NC * N_PAD, DH), jnp.float32)
    xs = xs.at[:N_NODES].set(emb_weight[:, :DH])
    xs = xs.at[N_PAD:N_PAD + N_NODES].set(emb_weight[:, DH:])
    tok4 = input.reshape(-1).astype(jnp.int32).reshape(NT * TB, TBK)
    b12 = jnp.concatenate([bias1[:DH], bias1[DH:], bias2[:DH], bias2[DH:]])
    out, _h1, _h2 = _run(xs, row3, col3, ew4, et4, tok4, b12)
    return jnp.concatenate([out[:T], out[T:]], axis=-1).reshape(B, L, D)
